# baseline (speedup 1.0000x reference)
import functools

import jax
import jax.numpy as jnp
from jax import lax
from jax.experimental import pallas as pl
from jax.experimental.pallas import tpu as pltpu
from jax.experimental.pallas import tpu_sc as plsc

N = 10000
F = 128
HID = 64
E = 320000
NT = 3
C = NT * HID
NC, NS = 2, 16
NW = NC * NS
EPW = E // NW
CH = C // NC
CHUNK = 80
NCHUNK = EPW // CHUNK
ESC = E // NS
NCHUNK_SC = ESC // CHUNK
SB = 10
NBLK = NCHUNK_SC // SB
CK = 128
RPSUB = 160
EPAD = CK * RPSUB * NS * NC // NC
NPAIR = RPSUB // 2
SUBA = 624
SUBN = 640
ZR = 40
RB = 1000
GRID = N // RB

_mesh = plsc.VectorSubcoreMesh(core_axis_name="c", subcore_axis_name="s")


@functools.partial(
    pl.kernel,
    out_type=jax.ShapeDtypeStruct((NC * N, 16), jnp.float32),
    mesh=_mesh,
    scratch_types=[
        pltpu.VMEM((NCHUNK, CHUNK), jnp.int32),
        pltpu.VMEM((CHUNK, 16), jnp.float32),
        pltpu.VMEM((SUBN, 16), jnp.float32),
        pltpu.VMEM_SHARED((N, 16), jnp.float32),
        pltpu.SemaphoreType.DMA,
    ],
    compiler_params=pltpu.CompilerParams(use_tc_tiling_on_sc=False),
)
def _sc_degree(dst_hbm, deg_hbm, dstbuf, ones_v, zbuf, acc, sem):
    c = lax.axis_index("c")
    s = lax.axis_index("s")
    w = c * NS + s

    @pl.loop(0, CHUNK)
    def _fill_ones(i):
        ones_v[i, :] = jnp.ones((16,), jnp.float32)

    @pl.loop(0, SUBN)
    def _fill_zero(i):
        zbuf[i, :] = jnp.zeros((16,), jnp.float32)

    pltpu.sync_copy(zbuf, acc.at[pl.ds(s * SUBA, SUBN)])
    pltpu.sync_copy(dst_hbm.at[pl.ds(w * NCHUNK, NCHUNK)], dstbuf)
    plsc.subcore_barrier()

    GK = 5

    @pl.loop(0, NCHUNK // GK)
    def _edges(i):
        descs = [
            pltpu.async_copy(ones_v, acc.at[dstbuf.at[i * GK + j]], sem,
                             add=True)
            for j in range(GK)
        ]
        for d in descs:
            d.wait()

    plsc.subcore_barrier()
    pltpu.sync_copy(acc.at[pl.ds(s * SUBA, SUBN)],
                    deg_hbm.at[pl.ds(c * N + s * SUBA, SUBN)])


@functools.partial(
    pl.kernel,
    out_type=jax.ShapeDtypeStruct((NC * N, CH), jnp.float32),
    mesh=_mesh,
    scratch_types=[
        pltpu.VMEM((RPSUB, CK), jnp.int32),
        pltpu.VMEM((RPSUB, CK), jnp.int32),
        pltpu.VMEM((CK, CH), jnp.float32),
        pltpu.VMEM((CK, CH), jnp.float32),
        pltpu.VMEM((ZR, CH), jnp.float32),
        pltpu.VMEM_SHARED((N + 8, CH), jnp.float32),
        pltpu.SemaphoreType.DMA,
        pltpu.SemaphoreType.DMA,
        pltpu.SemaphoreType.DMA,
        pltpu.SemaphoreType.DMA,
    ],
    compiler_params=pltpu.CompilerParams(use_tc_tiling_on_sc=False),
)
def _sc_spmm(p_hbm, src_hbm, dst_hbm, out_hbm, srcbuf, dstbuf, rows0, rows1,
             zbuf, acc, gs0, gs1, ss0, ss1):
    c = lax.axis_index("c")
    s = lax.axis_index("s")

    @pl.loop(0, ZR)
    def _fill_zero(i):
        @pl.loop(0, CH // 16)
        def _fz2(j):
            zbuf[i, pl.ds(j * 16, 16)] = jnp.zeros((16,), jnp.float32)

    @pl.loop(0, SUBN // ZR)
    def _clear(k):
        pltpu.sync_copy(zbuf, acc.at[pl.ds(s * SUBA + k * ZR, ZR)])

    @pl.when(s == 0)
    def _clear_trash():
        pltpu.sync_copy(zbuf.at[pl.ds(0, 8)], acc.at[pl.ds(N, 8)])

    pltpu.sync_copy(src_hbm.at[pl.ds((c * NS + s) * RPSUB, RPSUB)], srcbuf)
    pltpu.sync_copy(dst_hbm.at[pl.ds(s * RPSUB, RPSUB)], dstbuf)

    plsc.subcore_barrier()

    bufs = (rows0, rows1)
    gsems = (gs0, gs1)
    ssems = (ss0, ss1)

    def gather(row, k):
        pltpu.async_copy(p_hbm.at[srcbuf.at[row]], bufs[k], gsems[k])

    def gwait(k):
        pltpu.make_async_copy(p_hbm.at[srcbuf.at[0]], bufs[k], gsems[k]).wait()

    def scatter(row, k):
        pltpu.async_copy(bufs[k], acc.at[dstbuf.at[row]], ssems[k], add=True)

    def swait(k):
        pltpu.make_async_copy(p_hbm.at[srcbuf.at[0]], bufs[k], ssems[k]).wait()

    gather(0, 0)
    gather(1, 1)

    @pl.loop(0, NPAIR - 1)
    def _body(i):
        j = 2 * i
        gwait(0)
        scatter(j, 0)
        swait(0)
        gather(j + 2, 0)
        gwait(1)
        scatter(j + 1, 1)
        swait(1)
        gather(j + 3, 1)

    gwait(0)
    scatter(RPSUB - 2, 0)
    gwait(1)
    scatter(RPSUB - 1, 1)
    swait(0)
    swait(1)

    plsc.subcore_barrier()
    pltpu.sync_copy(acc.at[pl.ds(s * SUBA, SUBN)],
                    out_hbm.at[pl.ds(c * N + s * SUBA, SUBN)])


def _dinv_from(deg_ref):
    d = deg_ref[0, :, 0:1] + deg_ref[1, :, 0:1] + 1.0
    dinv = lax.rsqrt(d)
    return dinv, 1.0 / d


def _ka_body(xs_ref, w1_ref, deg_ref, p_ref, u_ref):
    dinv, _ = _dinv_from(deg_ref)
    w1 = w1_ref[...]
    u = jnp.concatenate(
        [jnp.dot(xs_ref[t], w1, preferred_element_type=jnp.float32)
         for t in range(NT)], axis=1)
    u_ref[...] = u
    p = u * dinv
    p_ref[0] = p[:, :CH]
    p_ref[1] = p[:, CH:]


def _kb_body(g_ref, u1_ref, deg_ref, w2_ref, b1_ref, p2_ref, u2_ref):
    dinv, dinv2 = _dinv_from(deg_ref)
    g = jnp.concatenate([g_ref[0], g_ref[1]], axis=1)
    h1 = jnp.maximum(g * dinv + u1_ref[...] * dinv2 + b1_ref[...], 0.0)
    w2 = w2_ref[...]
    u2 = jnp.concatenate(
        [jnp.dot(h1[:, t * HID:(t + 1) * HID], w2,
                 preferred_element_type=jnp.float32)
         for t in range(NT)], axis=1)
    u2_ref[...] = u2
    p2 = u2 * dinv
    p2_ref[0] = p2[:, :CH]
    p2_ref[1] = p2[:, CH:]


def _kc_body(g_ref, u2_ref, deg_ref, b2_ref, a10_ref, a11_ref, bcat_ref,
             cb1_ref, cb2_ref, fcwt_ref, fcb_ref, out_ref):
    dinv, dinv2 = _dinv_from(deg_ref)
    g = jnp.concatenate([g_ref[0], g_ref[1]], axis=1)
    h2 = jnp.maximum(g * dinv + u2_ref[...] * dinv2 + b2_ref[...], 0.0)
    c10 = jnp.maximum(
        jnp.dot(h2, a10_ref[...], preferred_element_type=jnp.float32)
        + cb1_ref[...], 0.0)
    c11 = jnp.maximum(
        jnp.dot(h2, a11_ref[...], preferred_element_type=jnp.float32)
        + cb1_ref[...], 0.0)
    cc = jnp.concatenate([c10, c11], axis=1)
    hl = jnp.maximum(
        jnp.dot(cc, bcat_ref[...], preferred_element_type=jnp.float32)
        + cb2_ref[...], 0.0)
    out_ref[...] = jnp.sum(hl * fcwt_ref[...], axis=1, keepdims=True) \
        + fcb_ref[0, 0]


def _row_spec(shape):
    return pl.BlockSpec(shape, lambda i: (i,) + (0,) * (len(shape) - 1))


def _full_spec(shape):
    return pl.BlockSpec(shape, lambda i: (0,) * len(shape))


_DEG_SPEC = pl.BlockSpec((2, RB, 16), lambda i: (0, i, 0))
_G_SPEC = pl.BlockSpec((2, RB, CH), lambda i: (0, i, 0))


_ka = pl.pallas_call(
    _ka_body,
    grid=(GRID,),
    in_specs=[pl.BlockSpec((NT, RB, F), lambda i: (0, i, 0)),
              _full_spec((F, HID)),
              _DEG_SPEC],
    out_specs=[pl.BlockSpec((2, RB, CH), lambda i: (0, i, 0)),
               _row_spec((RB, C))],
    out_shape=[jax.ShapeDtypeStruct((2, N, CH), jnp.float32),
               jax.ShapeDtypeStruct((N, C), jnp.float32)],
)

_kb = pl.pallas_call(
    _kb_body,
    grid=(GRID,),
    in_specs=[_G_SPEC,
              _row_spec((RB, C)),
              _DEG_SPEC,
              _full_spec((HID, HID)),
              _full_spec((1, C))],
    out_specs=[pl.BlockSpec((2, RB, CH), lambda i: (0, i, 0)),
               _row_spec((RB, C))],
    out_shape=[jax.ShapeDtypeStruct((2, N, CH), jnp.float32),
               jax.ShapeDtypeStruct((N, C), jnp.float32)],
)

_kc = pl.pallas_call(
    _kc_body,
    grid=(GRID,),
    in_specs=[_G_SPEC,
              _row_spec((RB, C)),
              _DEG_SPEC,
              _full_spec((1, C)),
              _full_spec((C, 32)),
              _full_spec((C, 32)),
              _full_spec((2 * 32, 32)),
              _full_spec((1, 32)),
              _full_spec((1, 32)),
              _full_spec((1, 32)),
              _full_spec((1, 1))],
    out_specs=_row_spec((RB, 1)),
    out_shape=jax.ShapeDtypeStruct((N, 1), jnp.float32),
)


def kernel(x, edge_index, W1, b1, W2, b2, cw1, cb1, cw2, cb2, fcw, fcb):
    src = edge_index[0].astype(jnp.int32)
    dst = edge_index[1].astype(jnp.int32)
    xs = x[0, 12 - NT:]

    dst2d0 = dst.reshape(E // CHUNK, CHUNK)
    deg = _sc_degree(dst2d0).reshape(2, N, 16)

    p1, u1 = _ka(xs, W1, deg)
    srcp = jnp.concatenate([src, jnp.zeros((EPAD - E,), jnp.int32)])
    dstp = jnp.concatenate([dst, jnp.full((EPAD - E,), N, jnp.int32)])
    src2d = srcp.reshape(EPAD // CK, CK)
    srccat = jnp.concatenate([src2d, src2d + N], axis=0)
    dst2d = dstp.reshape(EPAD // CK, CK)
    g1 = _sc_spmm(p1.reshape(NC * N, CH), srccat, dst2d).reshape(2, N, CH)

    b1t = jnp.tile(b1, NT)[None, :]
    p2, u2 = _kb(g1, u1, deg, W2, b1t)
    g2 = _sc_spmm(p2.reshape(NC * N, CH), srccat, dst2d).reshape(2, N, CH)

    a0, a1, a2 = (cw1[:, :, k].T for k in range(3))
    a10 = jnp.concatenate([a0, a1, a2], axis=0)
    a11 = jnp.concatenate([jnp.zeros_like(a0), a0, a1], axis=0)
    bcat = jnp.concatenate([cw2[:, :, 0].T, cw2[:, :, 1].T], axis=0)
    b2t = jnp.tile(b2, NT)[None, :]
    out = _kc(g2, u2, deg, b2t, a10, a11, bcat,
              cb1[None, :], cb2[None, :], fcw.T, fcb.reshape(1, 1))
    return out.reshape(1, N)


# speedup vs baseline: 2.1820x; 2.1820x over previous
import functools

import jax
import jax.numpy as jnp
from jax import lax
from jax.experimental import pallas as pl
from jax.experimental.pallas import tpu as pltpu
from jax.experimental.pallas import tpu_sc as plsc

N = 10000
F = 128
HID = 64
E = 320000
NT = 3
C = NT * HID
NC, NS = 2, 16
NW = NC * NS
EPW = E // NW
CH = C // NC
CHUNK = 80
NCHUNK = EPW // CHUNK
ESC = E // NS
NCHUNK_SC = ESC // CHUNK
SB = 10
NBLK = NCHUNK_SC // SB
NPH = 2
PHCH = NCHUNK_SC // NPH
SUBA = 624
SUBN = 640
ZR = 40
RB = 1000
GRID = N // RB

_mesh = plsc.VectorSubcoreMesh(core_axis_name="c", subcore_axis_name="s")


@functools.partial(
    pl.kernel,
    out_type=jax.ShapeDtypeStruct((NC * N, 16), jnp.float32),
    mesh=_mesh,
    scratch_types=[
        pltpu.VMEM((NCHUNK, CHUNK), jnp.int32),
        pltpu.VMEM((CHUNK, 16), jnp.float32),
        pltpu.VMEM((SUBN, 16), jnp.float32),
        pltpu.VMEM_SHARED((N, 16), jnp.float32),
        pltpu.SemaphoreType.DMA,
    ],
    compiler_params=pltpu.CompilerParams(use_tc_tiling_on_sc=False),
)
def _sc_degree(dst_hbm, deg_hbm, dstbuf, ones_v, zbuf, acc, sem):
    c = lax.axis_index("c")
    s = lax.axis_index("s")
    w = c * NS + s

    @pl.loop(0, CHUNK)
    def _fill_ones(i):
        ones_v[i, :] = jnp.ones((16,), jnp.float32)

    @pl.loop(0, SUBN)
    def _fill_zero(i):
        zbuf[i, :] = jnp.zeros((16,), jnp.float32)

    pltpu.sync_copy(zbuf, acc.at[pl.ds(s * SUBA, SUBN)])
    pltpu.sync_copy(dst_hbm.at[pl.ds(w * NCHUNK, NCHUNK)], dstbuf)
    plsc.subcore_barrier()

    GK = 5

    @pl.loop(0, NCHUNK // GK)
    def _edges(i):
        descs = [
            pltpu.async_copy(ones_v, acc.at[dstbuf.at[i * GK + j]], sem,
                             add=True)
            for j in range(GK)
        ]
        for d in descs:
            d.wait()

    plsc.subcore_barrier()
    pltpu.sync_copy(acc.at[pl.ds(s * SUBA, SUBN)],
                    deg_hbm.at[pl.ds(c * N + s * SUBA, SUBN)])


@functools.partial(
    pl.kernel,
    out_type=jax.ShapeDtypeStruct((NC * N, CH), jnp.float32),
    mesh=_mesh,
    scratch_types=[
        pltpu.VMEM((PHCH, CHUNK), jnp.int32),
        pltpu.VMEM((PHCH, CHUNK), jnp.int32),
        pltpu.VMEM((CHUNK, CH), jnp.float32),
        pltpu.VMEM((CHUNK, CH), jnp.float32),
        pltpu.VMEM((CHUNK, CH), jnp.float32),
        pltpu.VMEM((CHUNK, CH), jnp.float32),
        pltpu.VMEM((CHUNK, CH), jnp.float32),
        pltpu.VMEM((ZR, CH), jnp.float32),
        pltpu.VMEM_SHARED((N, CH), jnp.float32),
        pltpu.SemaphoreType.DMA,
        pltpu.SemaphoreType.DMA,
        pltpu.SemaphoreType.DMA,
        pltpu.SemaphoreType.DMA,
        pltpu.SemaphoreType.DMA,
        pltpu.SemaphoreType.DMA,
        pltpu.SemaphoreType.DMA,
        pltpu.SemaphoreType.DMA,
        pltpu.SemaphoreType.DMA,
        pltpu.SemaphoreType.DMA,
    ],
    compiler_params=pltpu.CompilerParams(use_tc_tiling_on_sc=False),
)
def _sc_spmm(p_hbm, src_hbm, dst_hbm, out_hbm, srcbuf, dstbuf, r0, r1, r2, r3,
             r4, zbuf, acc, g0, g1, g2, g3, g4, s0, s1, s2, s3, s4):
    c = lax.axis_index("c")
    s = lax.axis_index("s")

    @pl.loop(0, ZR)
    def _fill_zero(i):
        @pl.loop(0, CH // 16)
        def _fz2(j):
            zbuf[i, pl.ds(j * 16, 16)] = jnp.zeros((16,), jnp.float32)

    @pl.loop(0, SUBN // ZR)
    def _clear(k):
        pltpu.sync_copy(zbuf, acc.at[pl.ds(s * SUBA + k * ZR, ZR)])

    plsc.subcore_barrier()

    bufs = (r0, r1, r2, r3, r4)
    gsems = (g0, g1, g2, g3, g4)
    ssems = (s0, s1, s2, s3, s4)
    NB = 5

    def gather(row, k):
        pltpu.async_copy(p_hbm.at[srcbuf.at[row]], bufs[k], gsems[k])

    def gwait(k):
        pltpu.make_async_copy(p_hbm.at[srcbuf.at[0]], bufs[k], gsems[k]).wait()

    def scatter(row, k):
        pltpu.async_copy(bufs[k], acc.at[dstbuf.at[row]], ssems[k], add=True)

    def swait(k):
        pltpu.make_async_copy(p_hbm.at[srcbuf.at[0]], bufs[k], ssems[k]).wait()

    for ph in range(NPH):
        pltpu.sync_copy(
            src_hbm.at[pl.ds((c * NS + s) * NCHUNK_SC + ph * PHCH, PHCH)],
            srcbuf)
        pltpu.sync_copy(
            dst_hbm.at[pl.ds(s * NCHUNK_SC + ph * PHCH, PHCH)], dstbuf)

        for k in range(NB):
            gather(k, k)

        @pl.loop(0, PHCH // NB - 1)
        def _body(i):
            j = NB * i
            for k in range(NB):
                gwait(k)
                scatter(j + k, k)
            for k in range(NB):
                swait(k)
                gather(j + NB + k, k)

        j = PHCH - NB
        for k in range(NB):
            gwait(k)
            scatter(j + k, k)
        for k in range(NB):
            swait(k)

    plsc.subcore_barrier()
    pltpu.sync_copy(acc.at[pl.ds(s * SUBA, SUBN)],
                    out_hbm.at[pl.ds(c * N + s * SUBA, SUBN)])


def _dinv_from(deg_ref):
    d = deg_ref[0, :, 0:1] + deg_ref[1, :, 0:1] + 1.0
    dinv = lax.rsqrt(d)
    return dinv, 1.0 / d


def _ka_body(xs_ref, w1_ref, deg_ref, p_ref, u_ref):
    dinv, _ = _dinv_from(deg_ref)
    w1 = w1_ref[...]
    u = jnp.concatenate(
        [jnp.dot(xs_ref[t], w1, preferred_element_type=jnp.float32)
         for t in range(NT)], axis=1)
    u_ref[...] = u
    p = u * dinv
    p_ref[0] = p[:, :CH]
    p_ref[1] = p[:, CH:]


def _kb_body(g_ref, u1_ref, deg_ref, w2_ref, b1_ref, p2_ref, u2_ref):
    dinv, dinv2 = _dinv_from(deg_ref)
    g = jnp.concatenate([g_ref[0], g_ref[1]], axis=1)
    h1 = jnp.maximum(g * dinv + u1_ref[...] * dinv2 + b1_ref[...], 0.0)
    w2 = w2_ref[...]
    u2 = jnp.concatenate(
        [jnp.dot(h1[:, t * HID:(t + 1) * HID], w2,
                 preferred_element_type=jnp.float32)
         for t in range(NT)], axis=1)
    u2_ref[...] = u2
    p2 = u2 * dinv
    p2_ref[0] = p2[:, :CH]
    p2_ref[1] = p2[:, CH:]


def _kc_body(g_ref, u2_ref, deg_ref, b2_ref, a10_ref, a11_ref, bcat_ref,
             cb1_ref, cb2_ref, fcwt_ref, fcb_ref, out_ref):
    dinv, dinv2 = _dinv_from(deg_ref)
    g = jnp.concatenate([g_ref[0], g_ref[1]], axis=1)
    h2 = jnp.maximum(g * dinv + u2_ref[...] * dinv2 + b2_ref[...], 0.0)
    c10 = jnp.maximum(
        jnp.dot(h2, a10_ref[...], preferred_element_type=jnp.float32)
        + cb1_ref[...], 0.0)
    c11 = jnp.maximum(
        jnp.dot(h2, a11_ref[...], preferred_element_type=jnp.float32)
        + cb1_ref[...], 0.0)
    cc = jnp.concatenate([c10, c11], axis=1)
    hl = jnp.maximum(
        jnp.dot(cc, bcat_ref[...], preferred_element_type=jnp.float32)
        + cb2_ref[...], 0.0)
    out_ref[...] = jnp.sum(hl * fcwt_ref[...], axis=1, keepdims=True) \
        + fcb_ref[0, 0]


def _row_spec(shape):
    return pl.BlockSpec(shape, lambda i: (i,) + (0,) * (len(shape) - 1))


def _full_spec(shape):
    return pl.BlockSpec(shape, lambda i: (0,) * len(shape))


_DEG_SPEC = pl.BlockSpec((2, RB, 16), lambda i: (0, i, 0))
_G_SPEC = pl.BlockSpec((2, RB, CH), lambda i: (0, i, 0))


_ka = pl.pallas_call(
    _ka_body,
    grid=(GRID,),
    in_specs=[pl.BlockSpec((NT, RB, F), lambda i: (0, i, 0)),
              _full_spec((F, HID)),
              _DEG_SPEC],
    out_specs=[pl.BlockSpec((2, RB, CH), lambda i: (0, i, 0)),
               _row_spec((RB, C))],
    out_shape=[jax.ShapeDtypeStruct((2, N, CH), jnp.float32),
               jax.ShapeDtypeStruct((N, C), jnp.float32)],
)

_kb = pl.pallas_call(
    _kb_body,
    grid=(GRID,),
    in_specs=[_G_SPEC,
              _row_spec((RB, C)),
              _DEG_SPEC,
              _full_spec((HID, HID)),
              _full_spec((1, C))],
    out_specs=[pl.BlockSpec((2, RB, CH), lambda i: (0, i, 0)),
               _row_spec((RB, C))],
    out_shape=[jax.ShapeDtypeStruct((2, N, CH), jnp.float32),
               jax.ShapeDtypeStruct((N, C), jnp.float32)],
)

_kc = pl.pallas_call(
    _kc_body,
    grid=(GRID,),
    in_specs=[_G_SPEC,
              _row_spec((RB, C)),
              _DEG_SPEC,
              _full_spec((1, C)),
              _full_spec((C, 32)),
              _full_spec((C, 32)),
              _full_spec((2 * 32, 32)),
              _full_spec((1, 32)),
              _full_spec((1, 32)),
              _full_spec((1, 32)),
              _full_spec((1, 1))],
    out_specs=_row_spec((RB, 1)),
    out_shape=jax.ShapeDtypeStruct((N, 1), jnp.float32),
)


def kernel(x, edge_index, W1, b1, W2, b2, cw1, cb1, cw2, cb2, fcw, fcb):
    src = edge_index[0].astype(jnp.int32)
    dst = edge_index[1].astype(jnp.int32)
    xs = x[0, 12 - NT:]

    dst2d0 = dst.reshape(E // CHUNK, CHUNK)
    deg = _sc_degree(dst2d0).reshape(2, N, 16)

    p1, u1 = _ka(xs, W1, deg)
    src2d = src.reshape(E // CHUNK, CHUNK)
    srccat = jnp.concatenate([src2d, src2d + N], axis=0)
    dst2d = dst.reshape(E // CHUNK, CHUNK)
    g1 = _sc_spmm(p1.reshape(NC * N, CH), srccat, dst2d).reshape(2, N, CH)

    b1t = jnp.tile(b1, NT)[None, :]
    p2, u2 = _kb(g1, u1, deg, W2, b1t)
    g2 = _sc_spmm(p2.reshape(NC * N, CH), srccat, dst2d).reshape(2, N, CH)

    a0, a1, a2 = (cw1[:, :, k].T for k in range(3))
    a10 = jnp.concatenate([a0, a1, a2], axis=0)
    a11 = jnp.concatenate([jnp.zeros_like(a0), a0, a1], axis=0)
    bcat = jnp.concatenate([cw2[:, :, 0].T, cw2[:, :, 1].T], axis=0)
    b2t = jnp.tile(b2, NT)[None, :]
    out = _kc(g2, u2, deg, b2t, a10, a11, bcat,
              cb1[None, :], cb2[None, :], fcw.T, fcb.reshape(1, 1))
    return out.reshape(1, N)
